# baseline (speedup 1.0000x reference)
import functools

import jax
import jax.numpy as jnp
from jax import lax
from jax.experimental import pallas as pl
from jax.experimental.pallas import tpu as pltpu
from jax.experimental.pallas import tpu_sc as plsc

N = 10000
F = 128
E = 320000
HC = 128
FH = F // 2

NC = 2
NS = 16
NW = NC * NS
B = 128
CH = 160
E_PAD = NS * B * CH
N_PAD = 10240
RT = N_PAD // NS
DEG_W = 16
DCH = E_PAD // (NW * B)

_mesh = lambda: plsc.VectorSubcoreMesh(
    core_axis_name="c", subcore_axis_name="s", num_cores=NC, num_subcores=NS)


@functools.partial(
    pl.kernel,
    out_type=jax.ShapeDtypeStruct((NC * N_PAD, DEG_W), jnp.float32),
    mesh=_mesh(),
    scratch_types=[
        pltpu.VMEM((DCH, B), jnp.int32),
        pltpu.VMEM((B, DEG_W), jnp.float32),
        pltpu.VMEM_SHARED((N_PAD, DEG_W), jnp.float32),
    ],
    compiler_params=pltpu.CompilerParams(use_tc_tiling_on_sc=False),
)
def _deg_kernel(c_hbm, ones_hbm, zdeg_hbm, out_hbm, cidx, obuf, deg_sh):
    cid = lax.axis_index("c")
    sid = lax.axis_index("s")
    wid = sid * NC + cid
    pltpu.sync_copy(zdeg_hbm.at[pl.ds(sid * RT, RT)],
                    deg_sh.at[pl.ds(sid * RT, RT)])
    pltpu.sync_copy(c_hbm.at[pl.ds(wid * DCH, DCH)], cidx)
    pltpu.sync_copy(ones_hbm, obuf)
    plsc.subcore_barrier()

    def step(g, carry):
        pltpu.sync_copy(obuf, deg_sh.at[cidx.at[g]], add=True)
        return carry

    lax.fori_loop(0, DCH, step, 0)
    plsc.subcore_barrier()
    pltpu.sync_copy(deg_sh.at[pl.ds(sid * RT, RT)],
                    out_hbm.at[pl.ds(cid * N_PAD + sid * RT, RT)])


@functools.partial(
    pl.kernel,
    out_type=jax.ShapeDtypeStruct((NC * N_PAD, FH), jnp.float32),
    mesh=_mesh(),
    scratch_types=[
        pltpu.VMEM((2, 2, B), jnp.int32),
        pltpu.VMEM((2, 2, B), jnp.int32),
        pltpu.VMEM((B, FH), jnp.float32),
        pltpu.VMEM((B, FH), jnp.float32),
        pltpu.VMEM_SHARED((N_PAD, FH), jnp.float32),
        pltpu.VMEM_SHARED((N_PAD, FH), jnp.float32),
        pltpu.SemaphoreType.DMA,
        pltpu.SemaphoreType.DMA,
        pltpu.SemaphoreType.DMA,
        pltpu.SemaphoreType.DMA,
    ],
    compiler_params=pltpu.CompilerParams(use_tc_tiling_on_sc=False),
)
def _prop_kernel(xp2_hbm, rc_hbm, zf_hbm, out_hbm,
                 ia4, ib4, rows_a, rows_b, xph_sh, acc_sh,
                 sem_a, sem_b, sem_ia, sem_ib):
    cid = lax.axis_index("c")
    sid = lax.axis_index("s")
    base = sid * CH
    pltpu.sync_copy(xp2_hbm.at[pl.ds(sid * RT, RT), pl.ds(cid * FH, FH)],
                    xph_sh.at[pl.ds(sid * RT, RT)])
    pltpu.sync_copy(zf_hbm.at[pl.ds(sid * RT, RT)],
                    acc_sh.at[pl.ds(sid * RT, RT)])
    plsc.subcore_barrier()

    pltpu.async_copy(rc_hbm.at[pl.ds(base, 2)], ia4, sem_ia)
    pltpu.async_copy(rc_hbm.at[pl.ds(base + 2, 2)], ib4, sem_ib)
    NP = CH // 4

    def half(j, idx4, sem_i, off):
        g = base + 4 * j + off
        pltpu.make_async_copy(rc_hbm.at[pl.ds(g, 2)], idx4, sem_i).wait()
        pltpu.async_copy(xph_sh.at[idx4.at[0, 0]], rows_a, sem_a)
        pltpu.async_copy(xph_sh.at[idx4.at[1, 0]], rows_b, sem_b)
        pltpu.make_async_copy(xph_sh.at[idx4.at[0, 0]], rows_a, sem_a).wait()
        pltpu.sync_copy(rows_a, acc_sh.at[idx4.at[0, 1]], add=True)
        pltpu.make_async_copy(xph_sh.at[idx4.at[1, 0]], rows_b, sem_b).wait()
        pltpu.sync_copy(rows_b, acc_sh.at[idx4.at[1, 1]], add=True)

        @pl.when(j + 1 < NP)
        def _():
            pltpu.async_copy(rc_hbm.at[pl.ds(g + 4, 2)], idx4, sem_i)

    def step(j, carry):
        half(j, ia4, sem_ia, 0)
        half(j, ib4, sem_ib, 2)
        return carry

    lax.fori_loop(0, NP, step, 0)
    plsc.subcore_barrier()
    pltpu.sync_copy(acc_sh.at[pl.ds(sid * RT, RT)],
                    out_hbm.at[pl.ds(cid * N_PAD + sid * RT, RT)])


def _prep_body(deg_ref, x_ref, xp_ref):
    deg = deg_ref[:N, 0:1] + deg_ref[N_PAD:N_PAD + N, 0:1] + 2.0
    dinv = lax.rsqrt(deg)
    xp_ref[:N, :] = dinv * x_ref[...]
    xp_ref[N:, :] = jnp.zeros((N_PAD - N, F), jnp.float32)


def _prep_call(degp, x):
    return pl.pallas_call(
        _prep_body,
        out_shape=jax.ShapeDtypeStruct((N_PAD, F), jnp.float32),
    )(degp, x)


def _dense_body(acc_ref, deg_ref, x_ref, wz_ref, wlz_ref, bz_ref, blz_ref,
                wh_ref, wlh_ref, bh_ref, blh_ref, w2_ref, b2_ref, y_ref):
    deg = deg_ref[:N, 0:1] + deg_ref[N_PAD:N_PAD + N, 0:1] + 2.0
    dinv = lax.rsqrt(deg)
    s = jnp.concatenate([acc_ref[:N, :], acc_ref[N_PAD:N_PAD + N, :]], axis=1)
    px = dinv * s + (2.0 * dinv * dinv) * x_ref[...]
    az = jnp.dot(wz_ref[...], wlz_ref[:HC, :], preferred_element_type=jnp.float32)
    ah = jnp.dot(wh_ref[...], wlh_ref[:HC, :], preferred_element_type=jnp.float32)
    cz = jnp.dot(bz_ref[...], wlz_ref[:HC, :], preferred_element_type=jnp.float32) + blz_ref[...]
    ch = jnp.dot(bh_ref[...], wlh_ref[:HC, :], preferred_element_type=jnp.float32) + blh_ref[...]
    z = jax.nn.sigmoid(jnp.dot(px, az, preferred_element_type=jnp.float32) + cz)
    ht = jnp.tanh(jnp.dot(px, ah, preferred_element_type=jnp.float32) + ch)
    y_ref[...] = (jnp.dot((1.0 - z) * ht, w2_ref[...],
                          preferred_element_type=jnp.float32) + b2_ref[...])


def _dense_call(accp, degp, x, Wz, Wlz, bz, blz, Wh, Wlh, bh, blh, W2, b2):
    return pl.pallas_call(
        _dense_body,
        out_shape=jax.ShapeDtypeStruct((N, 1), jnp.float32),
    )(accp, degp, x, Wz, Wlz, bz, blz, Wh, Wlh, bh, blh, W2, b2)


@jax.jit
def _run(x, edge_index, Wz, bz, Wlz, blz, Wh, bh, Wlh, blh, W2, b2):
    row = edge_index[0]
    col = edge_index[1]
    padv = jnp.full((E_PAD - E,), N, jnp.int32)
    rpad = jnp.concatenate([row, padv]).reshape(NS * CH, B)
    cpad = jnp.concatenate([col, padv]).reshape(NS * CH, B)
    rc = jnp.stack([rpad, cpad], axis=1)
    ones16 = jnp.ones((B, DEG_W), jnp.float32)
    zdeg = jnp.zeros((N_PAD, DEG_W), jnp.float32)
    zfeat = jnp.zeros((N_PAD, FH), jnp.float32)

    degp = _deg_kernel(cpad.reshape(NW * DCH, B), ones16, zdeg)
    xp2 = _prep_call(degp, x)
    accp = _prop_kernel(xp2, rc, zfeat)
    return _dense_call(accp, degp, x, Wz, Wlz,
                       bz.reshape(1, HC), blz.reshape(1, HC),
                       Wh, Wlh, bh.reshape(1, HC), blh.reshape(1, HC),
                       W2, b2.reshape(1, 1))


def kernel(x, edge_index, Wz, bz, Wlz, blz, Wr, br, Wlr, blr, Wh, bh, Wlh,
           blh, W2, b2):
    return _run(x, edge_index, Wz, bz, Wlz, blz, Wh, bh, Wlh, blh, W2, b2)


# speedup vs baseline: 1.0616x; 1.0616x over previous
import functools

import jax
import jax.numpy as jnp
from jax import lax
from jax.experimental import pallas as pl
from jax.experimental.pallas import tpu as pltpu
from jax.experimental.pallas import tpu_sc as plsc

N = 10000
F = 128
E = 320000
HC = 128
FH = F // 2

NC = 2
NS = 16
NW = NC * NS
B = 128
CH = 160
E_PAD = NS * B * CH
N_PAD = 10240
RT = N_PAD // NS
DEG_W = 16
DCH = E_PAD // (NW * B)

_mesh = lambda: plsc.VectorSubcoreMesh(
    core_axis_name="c", subcore_axis_name="s", num_cores=NC, num_subcores=NS)


@functools.partial(
    pl.kernel,
    out_type=jax.ShapeDtypeStruct((NC * N_PAD, DEG_W), jnp.float32),
    mesh=_mesh(),
    scratch_types=[
        pltpu.VMEM((DCH * B,), jnp.int32),
        pltpu.VMEM((B, DEG_W), jnp.float32),
        pltpu.VMEM_SHARED((N_PAD, DEG_W), jnp.float32),
    ],
    compiler_params=pltpu.CompilerParams(use_tc_tiling_on_sc=False),
)
def _deg_kernel(ei_hbm, ones_hbm, zdeg_hbm, out_hbm, cidx, obuf, deg_sh):
    cid = lax.axis_index("c")
    sid = lax.axis_index("s")
    wid = sid * NC + cid
    ew = E // NW
    pltpu.sync_copy(zdeg_hbm.at[pl.ds(sid * RT, RT)],
                    deg_sh.at[pl.ds(sid * RT, RT)])
    for k in range(ew // 16, DCH * B // 16):
        cidx[pl.ds(16 * k, 16)] = jnp.full((16,), N, jnp.int32)
    pltpu.sync_copy(ei_hbm.at[1, pl.ds(wid * ew, ew)], cidx.at[pl.ds(0, ew)])
    pltpu.sync_copy(ones_hbm, obuf)
    plsc.subcore_barrier()

    def step(g, carry):
        pltpu.sync_copy(obuf, deg_sh.at[cidx.at[pl.ds(g * B, B)]], add=True)
        return carry

    lax.fori_loop(0, DCH, step, 0)
    plsc.subcore_barrier()
    pltpu.sync_copy(deg_sh.at[pl.ds(sid * RT, RT)],
                    out_hbm.at[pl.ds(cid * N_PAD + sid * RT, RT)])


@functools.partial(
    pl.kernel,
    out_type=jax.ShapeDtypeStruct((NC * N_PAD, FH), jnp.float32),
    mesh=_mesh(),
    scratch_types=[
        pltpu.VMEM((2, B), jnp.int32),
        pltpu.VMEM((2, B), jnp.int32),
        pltpu.VMEM((B, FH), jnp.float32),
        pltpu.VMEM((B, FH), jnp.float32),
        pltpu.VMEM_SHARED((N_PAD, FH), jnp.float32),
        pltpu.VMEM_SHARED((N_PAD, FH), jnp.float32),
        pltpu.SemaphoreType.DMA,
        pltpu.SemaphoreType.DMA,
    ],
    compiler_params=pltpu.CompilerParams(use_tc_tiling_on_sc=False),
)
def _prop_kernel(xp2_hbm, rc_hbm, zf_hbm, out_hbm,
                 idx_a, idx_b, rows_a, rows_b, xph_sh, acc_sh, sem_a, sem_b):
    cid = lax.axis_index("c")
    sid = lax.axis_index("s")
    base = sid * CH
    pltpu.sync_copy(xp2_hbm.at[pl.ds(sid * RT, RT), pl.ds(cid * FH, FH)],
                    xph_sh.at[pl.ds(sid * RT, RT)])
    pltpu.sync_copy(zf_hbm.at[pl.ds(sid * RT, RT)],
                    acc_sh.at[pl.ds(sid * RT, RT)])
    plsc.subcore_barrier()

    pltpu.sync_copy(rc_hbm.at[base], idx_a)
    pltpu.async_copy(xph_sh.at[idx_a.at[0]], rows_a, sem_a)

    def step(i, carry):
        g = base + 2 * i
        pltpu.sync_copy(rc_hbm.at[g + 1], idx_b)
        pltpu.async_copy(xph_sh.at[idx_b.at[0]], rows_b, sem_b)
        pltpu.make_async_copy(xph_sh.at[idx_a.at[0]], rows_a, sem_a).wait()
        pltpu.sync_copy(rows_a, acc_sh.at[idx_a.at[1]], add=True)

        @pl.when(i + 1 < CH // 2)
        def _():
            pltpu.sync_copy(rc_hbm.at[g + 2], idx_a)
            pltpu.async_copy(xph_sh.at[idx_a.at[0]], rows_a, sem_a)

        pltpu.make_async_copy(xph_sh.at[idx_b.at[0]], rows_b, sem_b).wait()
        pltpu.sync_copy(rows_b, acc_sh.at[idx_b.at[1]], add=True)
        return carry

    lax.fori_loop(0, CH // 2, step, 0)
    plsc.subcore_barrier()
    pltpu.sync_copy(acc_sh.at[pl.ds(sid * RT, RT)],
                    out_hbm.at[pl.ds(cid * N_PAD + sid * RT, RT)])


def _prep_body(deg_ref, x_ref, xp_ref):
    deg = deg_ref[:N, 0:1] + deg_ref[N_PAD:N_PAD + N, 0:1] + 2.0
    dinv = lax.rsqrt(deg)
    xp_ref[:N, :] = dinv * x_ref[...]
    xp_ref[N:, :] = jnp.zeros((N_PAD - N, F), jnp.float32)


def _prep_call(degp, x):
    return pl.pallas_call(
        _prep_body,
        out_shape=jax.ShapeDtypeStruct((N_PAD, F), jnp.float32),
    )(degp, x)


def _dense_body(acc_ref, deg_ref, x_ref, wz_ref, wlz_ref, bz_ref, blz_ref,
                wh_ref, wlh_ref, bh_ref, blh_ref, w2_ref, b2_ref, y_ref):
    deg = deg_ref[:N, 0:1] + deg_ref[N_PAD:N_PAD + N, 0:1] + 2.0
    dinv = lax.rsqrt(deg)
    s = jnp.concatenate([acc_ref[:N, :], acc_ref[N_PAD:N_PAD + N, :]], axis=1)
    px = dinv * s + (2.0 * dinv * dinv) * x_ref[...]
    az = jnp.dot(wz_ref[...], wlz_ref[:HC, :], preferred_element_type=jnp.float32)
    ah = jnp.dot(wh_ref[...], wlh_ref[:HC, :], preferred_element_type=jnp.float32)
    cz = jnp.dot(bz_ref[...], wlz_ref[:HC, :], preferred_element_type=jnp.float32) + blz_ref[...]
    ch = jnp.dot(bh_ref[...], wlh_ref[:HC, :], preferred_element_type=jnp.float32) + blh_ref[...]
    z = jax.nn.sigmoid(jnp.dot(px, az, preferred_element_type=jnp.float32) + cz)
    ht = jnp.tanh(jnp.dot(px, ah, preferred_element_type=jnp.float32) + ch)
    y_ref[...] = (jnp.dot((1.0 - z) * ht, w2_ref[...],
                          preferred_element_type=jnp.float32) + b2_ref[...])


def _dense_call(accp, degp, x, Wz, Wlz, bz, blz, Wh, Wlh, bh, blh, W2, b2):
    return pl.pallas_call(
        _dense_body,
        out_shape=jax.ShapeDtypeStruct((N, 1), jnp.float32),
    )(accp, degp, x, Wz, Wlz, bz, blz, Wh, Wlh, bh, blh, W2, b2)


@jax.jit
def _run(x, edge_index, Wz, bz, Wlz, blz, Wh, bh, Wlh, blh, W2, b2):
    row = edge_index[0]
    col = edge_index[1]
    padv = jnp.full((E_PAD - E,), N, jnp.int32)
    rpad = jnp.concatenate([row, padv]).reshape(NS * CH, B)
    cpad = jnp.concatenate([col, padv]).reshape(NS * CH, B)
    rc = jnp.stack([rpad, cpad], axis=1)
    ones16 = jnp.ones((B, DEG_W), jnp.float32)
    zdeg = jnp.zeros((N_PAD, DEG_W), jnp.float32)
    zfeat = jnp.zeros((N_PAD, FH), jnp.float32)

    degp = _deg_kernel(edge_index, ones16, zdeg)
    xp2 = _prep_call(degp, x)
    accp = _prop_kernel(xp2, rc, zfeat)
    return _dense_call(accp, degp, x, Wz, Wlz,
                       bz.reshape(1, HC), blz.reshape(1, HC),
                       Wh, Wlh, bh.reshape(1, HC), blh.reshape(1, HC),
                       W2, b2.reshape(1, 1))


def kernel(x, edge_index, Wz, bz, Wlz, blz, Wr, br, Wlr, blr, Wh, bh, Wlh,
           blh, W2, b2):
    return _run(x, edge_index, Wz, bz, Wlz, blz, Wh, bh, Wlh, blh, W2, b2)


# speedup vs baseline: 1.1076x; 1.0433x over previous
import functools

import jax
import jax.numpy as jnp
from jax import lax
from jax.experimental import pallas as pl
from jax.experimental.pallas import tpu as pltpu
from jax.experimental.pallas import tpu_sc as plsc

N = 10000
F = 128
E = 320000
HC = 128
FH = F // 2

NC = 2
NS = 16
NW = NC * NS
B = 128
CH = 160
E_PAD = NS * B * CH
N_PAD = 10240
RT = N_PAD // NS
DEG_W = 16
DCH = E_PAD // (NW * B)

_mesh = lambda: plsc.VectorSubcoreMesh(
    core_axis_name="c", subcore_axis_name="s", num_cores=NC, num_subcores=NS)


@functools.partial(
    pl.kernel,
    out_type=jax.ShapeDtypeStruct((NC * N_PAD, DEG_W), jnp.float32),
    mesh=_mesh(),
    scratch_types=[
        pltpu.VMEM((DCH * B,), jnp.int32),
        pltpu.VMEM((B, DEG_W), jnp.float32),
        pltpu.VMEM_SHARED((N_PAD, DEG_W), jnp.float32),
    ],
    compiler_params=pltpu.CompilerParams(use_tc_tiling_on_sc=False),
)
def _deg_kernel(ei_hbm, ones_hbm, zdeg_hbm, out_hbm, cidx, obuf, deg_sh):
    cid = lax.axis_index("c")
    sid = lax.axis_index("s")
    wid = sid * NC + cid
    ew = E // NW
    pltpu.sync_copy(zdeg_hbm.at[pl.ds(sid * RT, RT)],
                    deg_sh.at[pl.ds(sid * RT, RT)])
    for k in range(ew // 16, DCH * B // 16):
        cidx[pl.ds(16 * k, 16)] = jnp.full((16,), N, jnp.int32)
    pltpu.sync_copy(ei_hbm.at[pl.ds(E + wid * ew, ew)], cidx.at[pl.ds(0, ew)])
    pltpu.sync_copy(ones_hbm, obuf)
    plsc.subcore_barrier()

    def step(g, carry):
        pltpu.sync_copy(obuf, deg_sh.at[cidx.at[pl.ds(g * B, B)]], add=True)
        return carry

    lax.fori_loop(0, DCH, step, 0)
    plsc.subcore_barrier()
    pltpu.sync_copy(deg_sh.at[pl.ds(sid * RT, RT)],
                    out_hbm.at[pl.ds(cid * N_PAD + sid * RT, RT)])


@functools.partial(
    pl.kernel,
    out_type=jax.ShapeDtypeStruct((N_PAD, F), jnp.float32),
    mesh=_mesh(),
    scratch_types=[
        pltpu.VMEM((2, B), jnp.int32),
        pltpu.VMEM((2, B), jnp.int32),
        pltpu.VMEM((B, FH), jnp.float32),
        pltpu.VMEM((B, FH), jnp.float32),
        pltpu.VMEM_SHARED((N_PAD, FH), jnp.float32),
        pltpu.VMEM_SHARED((N_PAD, FH), jnp.float32),
        pltpu.SemaphoreType.DMA,
        pltpu.SemaphoreType.DMA,
    ],
    compiler_params=pltpu.CompilerParams(use_tc_tiling_on_sc=False),
)
def _prop_kernel(xp2_hbm, rc_hbm, zf_hbm, out_hbm,
                 idx_a, idx_b, rows_a, rows_b, xph_sh, acc_sh, sem_a, sem_b):
    cid = lax.axis_index("c")
    sid = lax.axis_index("s")
    base = sid * CH
    pltpu.sync_copy(xp2_hbm.at[pl.ds(sid * RT, RT), pl.ds(cid * FH, FH)],
                    xph_sh.at[pl.ds(sid * RT, RT)])
    pltpu.sync_copy(zf_hbm.at[pl.ds(sid * RT, RT)],
                    acc_sh.at[pl.ds(sid * RT, RT)])
    plsc.subcore_barrier()

    pltpu.sync_copy(rc_hbm.at[base], idx_a)
    pltpu.async_copy(xph_sh.at[idx_a.at[0]], rows_a, sem_a)

    def step(i, carry):
        g = base + 2 * i
        pltpu.sync_copy(rc_hbm.at[g + 1], idx_b)
        pltpu.async_copy(xph_sh.at[idx_b.at[0]], rows_b, sem_b)
        pltpu.make_async_copy(xph_sh.at[idx_a.at[0]], rows_a, sem_a).wait()
        pltpu.sync_copy(rows_a, acc_sh.at[idx_a.at[1]], add=True)

        @pl.when(i + 1 < CH // 2)
        def _():
            pltpu.sync_copy(rc_hbm.at[g + 2], idx_a)
            pltpu.async_copy(xph_sh.at[idx_a.at[0]], rows_a, sem_a)

        pltpu.make_async_copy(xph_sh.at[idx_b.at[0]], rows_b, sem_b).wait()
        pltpu.sync_copy(rows_b, acc_sh.at[idx_b.at[1]], add=True)
        return carry

    lax.fori_loop(0, CH // 2, step, 0)
    plsc.subcore_barrier()
    pltpu.sync_copy(acc_sh.at[pl.ds(sid * RT, RT)],
                    out_hbm.at[pl.ds(sid * RT, RT), pl.ds(cid * FH, FH)])


def _prep_body(deg_ref, x_ref, xp_ref):
    deg = deg_ref[:N, 0:1] + deg_ref[N_PAD:N_PAD + N, 0:1] + 2.0
    dinv = lax.rsqrt(deg)
    xp_ref[:N, :] = dinv * x_ref[...]
    xp_ref[N:, :] = jnp.zeros((N_PAD - N, F), jnp.float32)


def _prep_call(degp, x):
    return pl.pallas_call(
        _prep_body,
        out_shape=jax.ShapeDtypeStruct((N_PAD, F), jnp.float32),
    )(degp, x)


def _dense_body(acc_ref, deg_ref, x_ref, wz_ref, wlz_ref, bz_ref, blz_ref,
                wh_ref, wlh_ref, bh_ref, blh_ref, w2_ref, b2_ref, y_ref):
    deg = deg_ref[:N, 0:1] + deg_ref[N_PAD:N_PAD + N, 0:1] + 2.0
    dinv = lax.rsqrt(deg)
    s = acc_ref[:N, :]
    px = dinv * s + (2.0 * dinv * dinv) * x_ref[...]
    az = jnp.dot(wz_ref[...], wlz_ref[:HC, :], preferred_element_type=jnp.float32)
    ah = jnp.dot(wh_ref[...], wlh_ref[:HC, :], preferred_element_type=jnp.float32)
    cz = jnp.dot(bz_ref[...], wlz_ref[:HC, :], preferred_element_type=jnp.float32) + blz_ref[...]
    ch = jnp.dot(bh_ref[...], wlh_ref[:HC, :], preferred_element_type=jnp.float32) + blh_ref[...]
    z = jax.nn.sigmoid(jnp.dot(px, az, preferred_element_type=jnp.float32) + cz)
    ht = jnp.tanh(jnp.dot(px, ah, preferred_element_type=jnp.float32) + ch)
    y_ref[...] = (jnp.dot((1.0 - z) * ht, w2_ref[...],
                          preferred_element_type=jnp.float32) + b2_ref[...])


def _dense_call(accp, degp, x, Wz, Wlz, bz, blz, Wh, Wlh, bh, blh, W2, b2):
    return pl.pallas_call(
        _dense_body,
        out_shape=jax.ShapeDtypeStruct((N, 1), jnp.float32),
    )(accp, degp, x, Wz, Wlz, bz, blz, Wh, Wlh, bh, blh, W2, b2)


@jax.jit
def _run(x, edge_index, Wz, bz, Wlz, blz, Wh, bh, Wlh, blh, W2, b2):
    row = edge_index[0]
    col = edge_index[1]
    padv = jnp.full((E_PAD - E,), N, jnp.int32)
    rpad = jnp.concatenate([row, padv]).reshape(NS * CH, B)
    cpad = jnp.concatenate([col, padv]).reshape(NS * CH, B)
    rc = jnp.stack([rpad, cpad], axis=1)
    ones16 = jnp.ones((B, DEG_W), jnp.float32)
    zdeg = jnp.zeros((N_PAD, DEG_W), jnp.float32)
    zfeat = jnp.zeros((N_PAD, FH), jnp.float32)

    degp = _deg_kernel(edge_index.reshape(2 * E), ones16, zdeg)
    xp2 = _prep_call(degp, x)
    accp = _prop_kernel(xp2, rc, zfeat)
    return _dense_call(accp, degp, x, Wz, Wlz,
                       bz.reshape(1, HC), blz.reshape(1, HC),
                       Wh, Wlh, bh.reshape(1, HC), blh.reshape(1, HC),
                       W2, b2.reshape(1, 1))


def kernel(x, edge_index, Wz, bz, Wlz, blz, Wr, br, Wlr, blr, Wh, bh, Wlh,
           blh, W2, b2):
    return _run(x, edge_index, Wz, bz, Wlz, blz, Wh, bh, Wlh, blh, W2, b2)
